# baseline (speedup 1.0000x reference)
import functools

import jax
import jax.numpy as jnp
import numpy as np
from jax import lax
from jax.experimental import pallas as pl
from jax.experimental.pallas import tpu as pltpu
from jax.experimental.pallas import tpu_sc as plsc

N = 10000
E = 320000
IN = 128
H = 4
C = 32
ED = 4
G = 16
D = H * C
EF = E + N

NC = 2
NS = 16
NW = NC * NS
CH = 128
N_PAD = 10240
SLAB = N_PAD // NS

W1 = 10496
E_PAD = W1 * NW
W2 = 10752
EF_PAD = W2 * NW
NCH1 = W1 // CH
NCH2 = W2 // CH


def _sc_mesh():
    return plsc.VectorSubcoreMesh(core_axis_name="c", subcore_axis_name="s",
                                  num_cores=NC, num_subcores=NS)


_SH = np.zeros((D, 16), np.float32)
_TS = np.zeros((16, D), np.float32)
for _c in range(D):
    _SH[_c, _c // C] = 1.0
    _TS[_c // C, _c] = 1.0
_CMASK = np.zeros((1, 16), np.float32)
_CMASK[0, :H] = 1.0


def _k0_body(x_ref, wl_ref, bl_ref, wr_ref, br_ref, xl_ref, xr_ref):
    xb = x_ref[...]
    xl_ref[...] = jnp.dot(xb, wl_ref[...], preferred_element_type=jnp.float32) + bl_ref[...]
    xr_ref[...] = jnp.dot(xb, wr_ref[...], preferred_element_type=jnp.float32) + br_ref[...]


def _dense_proj(x, Wl, bl, Wr, br):
    bk = 1000
    return pl.pallas_call(
        _k0_body,
        grid=(N // bk,),
        in_specs=[
            pl.BlockSpec((bk, IN), lambda i: (i, 0)),
            pl.BlockSpec((IN, D), lambda i: (0, 0)),
            pl.BlockSpec((1, D), lambda i: (0, 0)),
            pl.BlockSpec((IN, D), lambda i: (0, 0)),
            pl.BlockSpec((1, D), lambda i: (0, 0)),
        ],
        out_specs=[
            pl.BlockSpec((bk, D), lambda i: (i, 0)),
            pl.BlockSpec((bk, D), lambda i: (i, 0)),
        ],
        out_shape=[
            jax.ShapeDtypeStruct((N, D), jnp.float32),
            jax.ShapeDtypeStruct((N, D), jnp.float32),
        ],
    )(x, Wl, bl.reshape(1, D), Wr, br.reshape(1, D))


def _make_scatter_body(w, with_dep):
    nch = w // CH

    def body(*refs):
        if with_dep:
            (rows_hbm, dst_hbm, zn_hbm, _dep, out_hbm,
             idx_v, b0, b1, ls0, ls1, ss0, ss1, acc) = refs
        else:
            (rows_hbm, dst_hbm, zn_hbm, out_hbm,
             idx_v, b0, b1, ls0, ls1, ss0, ss1, acc) = refs
        c = lax.axis_index("c")
        s = lax.axis_index("s")
        wid = s * NC + c
        bufs = (b0, b1)
        lsems = (ls0, ls1)
        ssems = (ss0, ss1)
        pltpu.sync_copy(zn_hbm, b0)
        for r in range(SLAB // CH):
            pltpu.sync_copy(b0, acc.at[pl.ds(s * SLAB + r * CH, CH)])
        plsc.subcore_barrier()
        pltpu.sync_copy(dst_hbm.at[wid], idx_v)
        base = wid * w
        for p in range(2):
            pltpu.async_copy(rows_hbm.at[pl.ds(base + p * CH, CH)],
                             bufs[p], lsems[p])

        def step(j2, carry):
            for p in range(2):
                j = 2 * j2 + p
                pltpu.make_async_copy(rows_hbm.at[pl.ds(base, CH)],
                                      bufs[p], lsems[p]).wait()
                pltpu.async_copy(bufs[p], acc.at[idx_v.at[j]],
                                 ssems[p], add=True)
                pltpu.make_async_copy(bufs[p], acc.at[pl.ds(0, CH)],
                                      ssems[p]).wait()

                @pl.when(j + 2 < nch)
                def _():
                    pltpu.async_copy(rows_hbm.at[pl.ds(base + (j + 2) * CH, CH)],
                                     bufs[p], lsems[p])
            return carry

        lax.fori_loop(0, nch // 2, step, 0)
        plsc.subcore_barrier()
        for r in range(SLAB // CH):
            pltpu.sync_copy(acc.at[pl.ds(s * SLAB + r * CH, CH)], b0)
            pltpu.sync_copy(b0, out_hbm.at[c, pl.ds(s * SLAB + r * CH, CH)])

    return body


def _make_scatter_kernel(w, with_dep):
    return pl.kernel(
        _make_scatter_body(w, with_dep),
        out_type=jax.ShapeDtypeStruct((NC, N_PAD, D), jnp.float32),
        mesh=_sc_mesh(),
        scratch_types=[
            pltpu.VMEM((w // CH, CH), jnp.int32),
            pltpu.VMEM((CH, D), jnp.float32),
            pltpu.VMEM((CH, D), jnp.float32),
            pltpu.SemaphoreType.DMA,
            pltpu.SemaphoreType.DMA,
            pltpu.SemaphoreType.DMA,
            pltpu.SemaphoreType.DMA,
            pltpu.VMEM_SHARED((N_PAD, D), jnp.float32),
        ],
    )


def _k2_gather(xl_hbm, xr_hbm, src_hbm, dst_hbm, xlg_hbm, xrg_hbm,
               si_v, di_v, a0, a1, a2, b0, b1, b2,
               ga0, ga1, ga2, gb0, gb1, gb2, wa0, wa1, wa2, wb0, wb1, wb2):
    c = lax.axis_index("c")
    s = lax.axis_index("s")
    wid = s * NC + c
    abufs = (a0, a1, a2)
    bbufs = (b0, b1, b2)
    gsa = (ga0, ga1, ga2)
    gsb = (gb0, gb1, gb2)
    wsa = (wa0, wa1, wa2)
    wsb = (wb0, wb1, wb2)
    base = wid * W2
    pltpu.sync_copy(src_hbm.at[wid], si_v)
    pltpu.sync_copy(dst_hbm.at[wid], di_v)
    for p in range(3):
        pltpu.async_copy(xl_hbm.at[si_v.at[p]], abufs[p], gsa[p])
        pltpu.async_copy(xr_hbm.at[di_v.at[p]], bbufs[p], gsb[p])

    def step(j3, carry):
        for p in range(3):
            j = 3 * j3 + p
            off = base + j * CH
            pltpu.make_async_copy(xl_hbm.at[si_v.at[0]], abufs[p], gsa[p]).wait()
            pltpu.make_async_copy(xr_hbm.at[di_v.at[0]], bbufs[p], gsb[p]).wait()
            pltpu.async_copy(abufs[p], xlg_hbm.at[pl.ds(off, CH)], wsa[p])
            pltpu.async_copy(bbufs[p], xrg_hbm.at[pl.ds(off, CH)], wsb[p])
            pn = (p + 2) % 3

            @pl.when(j + 2 < NCH2)
            def _():
                @pl.when(j >= 1)
                def _():
                    pltpu.make_async_copy(abufs[pn], xlg_hbm.at[pl.ds(base, CH)],
                                          wsa[pn]).wait()
                    pltpu.make_async_copy(bbufs[pn], xrg_hbm.at[pl.ds(base, CH)],
                                          wsb[pn]).wait()
                pltpu.async_copy(xl_hbm.at[si_v.at[j + 2]], abufs[pn], gsa[pn])
                pltpu.async_copy(xr_hbm.at[di_v.at[j + 2]], bbufs[pn], gsb[pn])
        return carry

    lax.fori_loop(0, NCH2 // 3, step, 0)
    for p in range(3):
        pltpu.make_async_copy(abufs[p], xlg_hbm.at[pl.ds(base, CH)], wsa[p]).wait()
        pltpu.make_async_copy(bbufs[p], xrg_hbm.at[pl.ds(base, CH)], wsb[p]).wait()


@functools.lru_cache(maxsize=None)
def _sc_kernels():
    k1 = _make_scatter_kernel(W1, with_dep=False)
    k2 = pl.kernel(
        _k2_gather,
        out_type=(
            jax.ShapeDtypeStruct((EF_PAD, D), jnp.float32),
            jax.ShapeDtypeStruct((EF_PAD, D), jnp.float32),
        ),
        mesh=_sc_mesh(),
        scratch_types=(
            [pltpu.VMEM((NCH2, CH), jnp.int32)] * 2
            + [pltpu.VMEM((CH, D), jnp.float32)] * 6
            + [pltpu.SemaphoreType.DMA] * 12
        ),
    )
    k4a = _make_scatter_kernel(W2, with_dep=False)
    k4b = _make_scatter_kernel(W2, with_dep=True)
    return k1, k2, k4a, k4b


def _k3_body(xlg_ref, xrg_ref, ea8_ref, we8_ref, att_ref, sh_ref, ts_ref, cm_ref,
             contrib_ref, t128_ref):
    i = pl.program_id(0)
    xlg = xlg_ref[...]
    z = xlg + xrg_ref[...] + jnp.dot(ea8_ref[...], we8_ref[...],
                                     preferred_element_type=jnp.float32)
    m = jnp.maximum(z, 0.2 * z)
    p = m * att_ref[...]
    alpha16 = jnp.dot(p, sh_ref[...], preferred_element_type=jnp.float32)
    rows = i * 1024 + lax.broadcasted_iota(jnp.int32, (1024, 1), 0)
    valid = (rows < EF).astype(jnp.float32)
    t16 = jnp.exp(alpha16) * cm_ref[...] * valid
    t128 = jnp.dot(t16, ts_ref[...], preferred_element_type=jnp.float32)
    contrib_ref[...] = xlg * t128
    t128_ref[...] = t128


def _edge_math(xlg, xrg, eaf8, We8, att128, sh, ts, cmask):
    bk = 1024
    return pl.pallas_call(
        _k3_body,
        grid=(EF_PAD // bk,),
        in_specs=[
            pl.BlockSpec((bk, D), lambda i: (i, 0)),
            pl.BlockSpec((bk, D), lambda i: (i, 0)),
            pl.BlockSpec((bk, 8), lambda i: (i, 0)),
            pl.BlockSpec((8, D), lambda i: (0, 0)),
            pl.BlockSpec((1, D), lambda i: (0, 0)),
            pl.BlockSpec((D, 16), lambda i: (0, 0)),
            pl.BlockSpec((16, D), lambda i: (0, 0)),
            pl.BlockSpec((1, 16), lambda i: (0, 0)),
        ],
        out_specs=[
            pl.BlockSpec((bk, D), lambda i: (i, 0)),
            pl.BlockSpec((bk, D), lambda i: (i, 0)),
        ],
        out_shape=[
            jax.ShapeDtypeStruct((EF_PAD, D), jnp.float32),
            jax.ShapeDtypeStruct((EF_PAD, D), jnp.float32),
        ],
    )(xlg, xrg, eaf8, We8, att128, sh, ts, cmask)


def _k5_body(pa0_ref, pa1_ref, pb0_ref, pb1_ref, batch_ref, bias_ref,
             gs_ref, gc_ref):
    i = pl.program_id(0)
    num = pa0_ref[...] + pa1_ref[...]
    den = pb0_ref[...] + pb1_ref[...]
    node = jnp.maximum(num / (den + 1e-16) + bias_ref[...], 0.0)
    b = batch_ref[0]
    oh = (lax.broadcasted_iota(jnp.int32, (G, 400), 0) == b).astype(jnp.float32)
    gsb = jnp.dot(oh, node, preferred_element_type=jnp.float32)
    gcb = jnp.broadcast_to(jnp.sum(oh, axis=1, keepdims=True), (G, D))

    @pl.when(i == 0)
    def _():
        gs_ref[...] = jnp.zeros_like(gs_ref)
        gc_ref[...] = jnp.zeros_like(gc_ref)

    gs_ref[...] += gsb
    gc_ref[...] += gcb


def _pool(pa, pb, batch, bias):
    bk = 400
    return pl.pallas_call(
        _k5_body,
        grid=(N // bk,),
        in_specs=[
            pl.BlockSpec((bk, D), lambda i: (i, 0)),
            pl.BlockSpec((bk, D), lambda i: (i, 0)),
            pl.BlockSpec((bk, D), lambda i: (i, 0)),
            pl.BlockSpec((bk, D), lambda i: (i, 0)),
            pl.BlockSpec((1, 1, bk), lambda i: (i, 0, 0)),
            pl.BlockSpec((1, D), lambda i: (0, 0)),
        ],
        out_specs=[
            pl.BlockSpec((G, D), lambda i: (0, 0)),
            pl.BlockSpec((G, D), lambda i: (0, 0)),
        ],
        out_shape=[
            jax.ShapeDtypeStruct((G, D), jnp.float32),
            jax.ShapeDtypeStruct((G, D), jnp.float32),
        ],
    )(pa[0], pa[1], pb[0], pb[1], batch.reshape(N // bk, 1, bk),
      bias.reshape(1, D))


def kernel(x, edge_index, edge_attr, batch, Wl, bl, Wr, br, We, att, bias):
    f32 = jnp.float32
    src = edge_index[0]
    dst = edge_index[1]

    xl, xr = _dense_proj(x, Wl, bl, Wr, br)
    k1, k2, k4a, k4b = _sc_kernels()
    zn128 = jnp.zeros((CH, D), f32)

    ea128 = jnp.concatenate(
        [edge_attr, jnp.ones((E, 1), f32), jnp.zeros((E, D - ED - 1), f32)], axis=1)
    ea128 = jnp.pad(ea128, ((0, E_PAD - E), (0, 0)))
    dst_p = jnp.pad(dst, (0, E_PAD - E)).reshape(NW, NCH1, CH)
    part = k1(ea128, dst_p, zn128)
    s = part[0, :N] + part[1, :N]
    loop_attr = s[:, :ED] / jnp.clip(s[:, ED:ED + 1], 1.0, None)

    loop = jnp.arange(N, dtype=src.dtype)
    srcf = jnp.pad(jnp.concatenate([src, loop]),
                   (0, EF_PAD - EF)).reshape(NW, NCH2, CH)
    dstf = jnp.pad(jnp.concatenate([dst, loop]),
                   (0, EF_PAD - EF)).reshape(NW, NCH2, CH)
    xlg, xrg = k2(xl, xr, srcf, dstf)

    eaf8 = jnp.pad(jnp.concatenate([edge_attr, loop_attr], axis=0),
                   ((0, EF_PAD - EF), (0, 4)))
    We8 = jnp.pad(We, ((0, 4), (0, 0)))
    att128 = att.reshape(1, D)
    contrib, t128 = _edge_math(xlg, xrg, eaf8, We8, att128,
                               jnp.asarray(_SH), jnp.asarray(_TS),
                               jnp.asarray(_CMASK))

    pa = k4a(contrib, dstf, zn128)
    pb = k4b(t128, dstf, zn128, pa)

    gs, gc = _pool(pa, pb, batch, bias)
    return gs / jnp.clip(gc, 1.0, None)


# speedup vs baseline: 1.0345x; 1.0345x over previous
import functools

import jax
import jax.numpy as jnp
import numpy as np
from jax import lax
from jax.experimental import pallas as pl
from jax.experimental.pallas import tpu as pltpu
from jax.experimental.pallas import tpu_sc as plsc

N = 10000
E = 320000
IN = 128
H = 4
C = 32
ED = 4
G = 16
D = H * C
EF = E + N

NC = 2
NS = 16
NW = NC * NS
CH = 128
N_PAD = 10240
SLAB = N_PAD // NS

W1 = 10496
E_PAD = W1 * NW
W2 = 10752
EF_PAD = W2 * NW
NCH1 = W1 // CH
NCH2 = W2 // CH


def _sc_mesh():
    return plsc.VectorSubcoreMesh(core_axis_name="c", subcore_axis_name="s",
                                  num_cores=NC, num_subcores=NS)


_SH = np.zeros((D, 16), np.float32)
_TS = np.zeros((16, D), np.float32)
for _c in range(D):
    _SH[_c, _c // C] = 1.0
    _TS[_c // C, _c] = 1.0
_CMASK = np.zeros((1, 16), np.float32)
_CMASK[0, :H] = 1.0


def _k0_body(x_ref, wl_ref, bl_ref, wr_ref, br_ref, xl_ref, xr_ref):
    xb = x_ref[...]
    xl_ref[...] = jnp.dot(xb, wl_ref[...], preferred_element_type=jnp.float32) + bl_ref[...]
    xr_ref[...] = jnp.dot(xb, wr_ref[...], preferred_element_type=jnp.float32) + br_ref[...]


def _dense_proj(x, Wl, bl, Wr, br):
    bk = 1000
    return pl.pallas_call(
        _k0_body,
        grid=(N // bk,),
        in_specs=[
            pl.BlockSpec((bk, IN), lambda i: (i, 0)),
            pl.BlockSpec((IN, D), lambda i: (0, 0)),
            pl.BlockSpec((1, D), lambda i: (0, 0)),
            pl.BlockSpec((IN, D), lambda i: (0, 0)),
            pl.BlockSpec((1, D), lambda i: (0, 0)),
        ],
        out_specs=[
            pl.BlockSpec((bk, D), lambda i: (i, 0)),
            pl.BlockSpec((bk, D), lambda i: (i, 0)),
        ],
        out_shape=[
            jax.ShapeDtypeStruct((N, D), jnp.float32),
            jax.ShapeDtypeStruct((N, D), jnp.float32),
        ],
    )(x, Wl, bl.reshape(1, D), Wr, br.reshape(1, D))


def _make_scatter_body(w, with_dep):
    nch = w // CH

    def body(*refs):
        if with_dep:
            (rows_hbm, dst_hbm, zn_hbm, _dep, out_hbm,
             idx_v, b0, b1, ls0, ls1, ss0, ss1, acc) = refs
        else:
            (rows_hbm, dst_hbm, zn_hbm, out_hbm,
             idx_v, b0, b1, ls0, ls1, ss0, ss1, acc) = refs
        c = lax.axis_index("c")
        s = lax.axis_index("s")
        wid = s * NC + c
        bufs = (b0, b1)
        lsems = (ls0, ls1)
        ssems = (ss0, ss1)
        pltpu.sync_copy(zn_hbm, b0)
        for r in range(SLAB // CH):
            pltpu.sync_copy(b0, acc.at[pl.ds(s * SLAB + r * CH, CH)])
        plsc.subcore_barrier()
        pltpu.sync_copy(dst_hbm.at[wid], idx_v)
        base = wid * w
        for p in range(2):
            pltpu.async_copy(rows_hbm.at[pl.ds(base + p * CH, CH)],
                             bufs[p], lsems[p])

        def step(j2, carry):
            for p in range(2):
                j = 2 * j2 + p
                pltpu.make_async_copy(rows_hbm.at[pl.ds(base, CH)],
                                      bufs[p], lsems[p]).wait()
                pltpu.async_copy(bufs[p], acc.at[idx_v.at[j]],
                                 ssems[p], add=True)
                pltpu.make_async_copy(bufs[p], acc.at[pl.ds(0, CH)],
                                      ssems[p]).wait()

                @pl.when(j + 2 < nch)
                def _():
                    pltpu.async_copy(rows_hbm.at[pl.ds(base + (j + 2) * CH, CH)],
                                     bufs[p], lsems[p])
            return carry

        lax.fori_loop(0, nch // 2, step, 0)
        plsc.subcore_barrier()
        for r in range(SLAB // CH):
            pltpu.sync_copy(acc.at[pl.ds(s * SLAB + r * CH, CH)], b0)
            pltpu.sync_copy(b0, out_hbm.at[c, pl.ds(s * SLAB + r * CH, CH)])

    return body


def _make_scatter_kernel(w, with_dep):
    return pl.kernel(
        _make_scatter_body(w, with_dep),
        out_type=jax.ShapeDtypeStruct((NC, N_PAD, D), jnp.float32),
        mesh=_sc_mesh(),
        scratch_types=[
            pltpu.VMEM((w // CH, CH), jnp.int32),
            pltpu.VMEM((CH, D), jnp.float32),
            pltpu.VMEM((CH, D), jnp.float32),
            pltpu.SemaphoreType.DMA,
            pltpu.SemaphoreType.DMA,
            pltpu.SemaphoreType.DMA,
            pltpu.SemaphoreType.DMA,
            pltpu.VMEM_SHARED((N_PAD, D), jnp.float32),
        ],
    )


def _k2_gather(xl_hbm, xr_hbm, src_hbm, dst_hbm, xlg_hbm, xrg_hbm,
               si_v, di_v, a0, a1, b0, b1,
               ga0, ga1, gb0, gb1, wa0, wa1, wb0, wb1):
    c = lax.axis_index("c")
    s = lax.axis_index("s")
    wid = s * NC + c
    abufs = (a0, a1)
    bbufs = (b0, b1)
    gsa = (ga0, ga1)
    gsb = (gb0, gb1)
    wsa = (wa0, wa1)
    wsb = (wb0, wb1)
    base = wid * W2
    pltpu.sync_copy(src_hbm.at[wid], si_v)
    pltpu.sync_copy(dst_hbm.at[wid], di_v)
    for p in range(2):
        pltpu.async_copy(xl_hbm.at[si_v.at[p]], abufs[p], gsa[p])
        pltpu.async_copy(xr_hbm.at[di_v.at[p]], bbufs[p], gsb[p])

    def step(j2, carry):
        for p in range(2):
            j = 2 * j2 + p
            off = base + j * CH
            pltpu.make_async_copy(xl_hbm.at[si_v.at[0]], abufs[p], gsa[p]).wait()
            pltpu.make_async_copy(xr_hbm.at[di_v.at[0]], bbufs[p], gsb[p]).wait()
            pltpu.async_copy(abufs[p], xlg_hbm.at[pl.ds(off, CH)], wsa[p])
            pltpu.async_copy(bbufs[p], xrg_hbm.at[pl.ds(off, CH)], wsb[p])
            pltpu.make_async_copy(abufs[p], xlg_hbm.at[pl.ds(base, CH)], wsa[p]).wait()
            pltpu.make_async_copy(bbufs[p], xrg_hbm.at[pl.ds(base, CH)], wsb[p]).wait()

            @pl.when(j + 2 < NCH2)
            def _():
                pltpu.async_copy(xl_hbm.at[si_v.at[j + 2]], abufs[p], gsa[p])
                pltpu.async_copy(xr_hbm.at[di_v.at[j + 2]], bbufs[p], gsb[p])
        return carry

    lax.fori_loop(0, NCH2 // 2, step, 0)


@functools.lru_cache(maxsize=None)
def _sc_kernels():
    k1 = _make_scatter_kernel(W1, with_dep=False)
    k2 = pl.kernel(
        _k2_gather,
        out_type=(
            jax.ShapeDtypeStruct((EF_PAD, D), jnp.float32),
            jax.ShapeDtypeStruct((EF_PAD, D), jnp.float32),
        ),
        mesh=_sc_mesh(),
        scratch_types=(
            [pltpu.VMEM((NCH2, CH), jnp.int32)] * 2
            + [pltpu.VMEM((CH, D), jnp.float32)] * 4
            + [pltpu.SemaphoreType.DMA] * 8
        ),
    )
    k4a = _make_scatter_kernel(W2, with_dep=False)
    k4b = _make_scatter_kernel(W2, with_dep=True)
    return k1, k2, k4a, k4b


def _k3_body(xlg_ref, xrg_ref, ea8_ref, we8_ref, att_ref, sh_ref, ts_ref, cm_ref,
             contrib_ref, t128_ref):
    i = pl.program_id(0)
    xlg = xlg_ref[...]
    z = xlg + xrg_ref[...] + jnp.dot(ea8_ref[...], we8_ref[...],
                                     preferred_element_type=jnp.float32)
    m = jnp.maximum(z, 0.2 * z)
    p = m * att_ref[...]
    alpha16 = jnp.dot(p, sh_ref[...], preferred_element_type=jnp.float32)
    rows = i * 1024 + lax.broadcasted_iota(jnp.int32, (1024, 1), 0)
    valid = (rows < EF).astype(jnp.float32)
    t16 = jnp.exp(alpha16) * cm_ref[...] * valid
    t128 = jnp.dot(t16, ts_ref[...], preferred_element_type=jnp.float32)
    contrib_ref[...] = xlg * t128
    t128_ref[...] = t128


def _edge_math(xlg, xrg, eaf8, We8, att128, sh, ts, cmask):
    bk = 1024
    return pl.pallas_call(
        _k3_body,
        grid=(EF_PAD // bk,),
        in_specs=[
            pl.BlockSpec((bk, D), lambda i: (i, 0)),
            pl.BlockSpec((bk, D), lambda i: (i, 0)),
            pl.BlockSpec((bk, 8), lambda i: (i, 0)),
            pl.BlockSpec((8, D), lambda i: (0, 0)),
            pl.BlockSpec((1, D), lambda i: (0, 0)),
            pl.BlockSpec((D, 16), lambda i: (0, 0)),
            pl.BlockSpec((16, D), lambda i: (0, 0)),
            pl.BlockSpec((1, 16), lambda i: (0, 0)),
        ],
        out_specs=[
            pl.BlockSpec((bk, D), lambda i: (i, 0)),
            pl.BlockSpec((bk, D), lambda i: (i, 0)),
        ],
        out_shape=[
            jax.ShapeDtypeStruct((EF_PAD, D), jnp.float32),
            jax.ShapeDtypeStruct((EF_PAD, D), jnp.float32),
        ],
    )(xlg, xrg, eaf8, We8, att128, sh, ts, cmask)


def _k5_body(pa0_ref, pa1_ref, pb0_ref, pb1_ref, batch_ref, bias_ref,
             gs_ref, gc_ref):
    i = pl.program_id(0)
    num = pa0_ref[...] + pa1_ref[...]
    den = pb0_ref[...] + pb1_ref[...]
    node = jnp.maximum(num / (den + 1e-16) + bias_ref[...], 0.0)
    b = batch_ref[0]
    oh = (lax.broadcasted_iota(jnp.int32, (G, 400), 0) == b).astype(jnp.float32)
    gsb = jnp.dot(oh, node, preferred_element_type=jnp.float32)
    gcb = jnp.broadcast_to(jnp.sum(oh, axis=1, keepdims=True), (G, D))

    @pl.when(i == 0)
    def _():
        gs_ref[...] = jnp.zeros_like(gs_ref)
        gc_ref[...] = jnp.zeros_like(gc_ref)

    gs_ref[...] += gsb
    gc_ref[...] += gcb


def _pool(pa, pb, batch, bias):
    bk = 400
    return pl.pallas_call(
        _k5_body,
        grid=(N // bk,),
        in_specs=[
            pl.BlockSpec((bk, D), lambda i: (i, 0)),
            pl.BlockSpec((bk, D), lambda i: (i, 0)),
            pl.BlockSpec((bk, D), lambda i: (i, 0)),
            pl.BlockSpec((bk, D), lambda i: (i, 0)),
            pl.BlockSpec((1, 1, bk), lambda i: (i, 0, 0)),
            pl.BlockSpec((1, D), lambda i: (0, 0)),
        ],
        out_specs=[
            pl.BlockSpec((G, D), lambda i: (0, 0)),
            pl.BlockSpec((G, D), lambda i: (0, 0)),
        ],
        out_shape=[
            jax.ShapeDtypeStruct((G, D), jnp.float32),
            jax.ShapeDtypeStruct((G, D), jnp.float32),
        ],
    )(pa[0], pa[1], pb[0], pb[1], batch.reshape(N // bk, 1, bk),
      bias.reshape(1, D))


def kernel(x, edge_index, edge_attr, batch, Wl, bl, Wr, br, We, att, bias):
    f32 = jnp.float32
    src = edge_index[0]
    dst = edge_index[1]

    xl, xr = _dense_proj(x, Wl, bl, Wr, br)
    k1, k2, k4a, k4b = _sc_kernels()
    zn128 = jnp.zeros((CH, D), f32)

    ea128 = jnp.concatenate(
        [edge_attr, jnp.ones((E, 1), f32), jnp.zeros((E, D - ED - 1), f32)], axis=1)
    ea128 = jnp.pad(ea128, ((0, E_PAD - E), (0, 0)))
    dst_p = jnp.pad(dst, (0, E_PAD - E)).reshape(NW, NCH1, CH)
    part = k1(ea128, dst_p, zn128)
    s = part[0, :N] + part[1, :N]
    loop_attr = s[:, :ED] / jnp.clip(s[:, ED:ED + 1], 1.0, None)

    loop = jnp.arange(N, dtype=src.dtype)
    srcf = jnp.pad(jnp.concatenate([src, loop]),
                   (0, EF_PAD - EF)).reshape(NW, NCH2, CH)
    dstf = jnp.pad(jnp.concatenate([dst, loop]),
                   (0, EF_PAD - EF)).reshape(NW, NCH2, CH)
    xlg, xrg = k2(xl, xr, srcf, dstf)

    eaf8 = jnp.pad(jnp.concatenate([edge_attr, loop_attr], axis=0),
                   ((0, EF_PAD - EF), (0, 4)))
    We8 = jnp.pad(We, ((0, 4), (0, 0)))
    att128 = att.reshape(1, D)
    contrib, t128 = _edge_math(xlg, xrg, eaf8, We8, att128,
                               jnp.asarray(_SH), jnp.asarray(_TS),
                               jnp.asarray(_CMASK))

    pa = k4a(contrib, dstf, zn128)
    pb = k4b(t128, dstf, zn128, pa)

    gs, gc = _pool(pa, pb, batch, bias)
    return gs / jnp.clip(gc, 1.0, None)


# speedup vs baseline: 1.5502x; 1.4985x over previous
import functools

import jax
import jax.numpy as jnp
import numpy as np
from jax import lax
from jax.experimental import pallas as pl
from jax.experimental.pallas import tpu as pltpu
from jax.experimental.pallas import tpu_sc as plsc

N = 10000
E = 320000
IN = 128
H = 4
C = 32
ED = 4
G = 16
D = H * C
EF = E + N

NC = 2
NS = 16
NW = NC * NS
CH = 128
N_PAD = 10240
SLAB = N_PAD // NS

W1 = 10496
E_PAD = W1 * NW
W2 = 10752
EF_PAD = W2 * NW
NCH1 = W1 // CH
NCH2 = W2 // CH


def _sc_mesh():
    return plsc.VectorSubcoreMesh(core_axis_name="c", subcore_axis_name="s",
                                  num_cores=NC, num_subcores=NS)


_SH = np.zeros((D, 16), np.float32)
_TS = np.zeros((16, D), np.float32)
for _c in range(D):
    _SH[_c, _c // C] = 1.0
    _TS[_c // C, _c] = 1.0
_CMASK = np.zeros((1, 16), np.float32)
_CMASK[0, :H] = 1.0


def _k0_body(x_ref, wl_ref, bl_ref, wr_ref, br_ref, xl_ref, xr_ref):
    xb = x_ref[...]
    xl_ref[...] = jnp.dot(xb, wl_ref[...], preferred_element_type=jnp.float32) + bl_ref[...]
    xr_ref[...] = jnp.dot(xb, wr_ref[...], preferred_element_type=jnp.float32) + br_ref[...]


def _dense_proj(x, Wl, bl, Wr, br):
    bk = 1000
    return pl.pallas_call(
        _k0_body,
        grid=(N // bk,),
        in_specs=[
            pl.BlockSpec((bk, IN), lambda i: (i, 0)),
            pl.BlockSpec((IN, D), lambda i: (0, 0)),
            pl.BlockSpec((1, D), lambda i: (0, 0)),
            pl.BlockSpec((IN, D), lambda i: (0, 0)),
            pl.BlockSpec((1, D), lambda i: (0, 0)),
        ],
        out_specs=[
            pl.BlockSpec((bk, D), lambda i: (i, 0)),
            pl.BlockSpec((bk, D), lambda i: (i, 0)),
        ],
        out_shape=[
            jax.ShapeDtypeStruct((N, D), jnp.float32),
            jax.ShapeDtypeStruct((N, D), jnp.float32),
        ],
    )(x, Wl, bl.reshape(1, D), Wr, br.reshape(1, D))


def _make_scatter_body(w, with_dep):
    nch = w // CH

    def body(*refs):
        if with_dep:
            (rows_hbm, dst_hbm, zn_hbm, _dep, out_hbm,
             idx_v, b0, b1, ls0, ls1, ss0, ss1, acc) = refs
        else:
            (rows_hbm, dst_hbm, zn_hbm, out_hbm,
             idx_v, b0, b1, ls0, ls1, ss0, ss1, acc) = refs
        c = lax.axis_index("c")
        s = lax.axis_index("s")
        wid = s * NC + c
        bufs = (b0, b1)
        lsems = (ls0, ls1)
        ssems = (ss0, ss1)
        pltpu.sync_copy(zn_hbm, b0)
        for r in range(SLAB // CH):
            pltpu.sync_copy(b0, acc.at[pl.ds(s * SLAB + r * CH, CH)])
        plsc.subcore_barrier()
        pltpu.sync_copy(dst_hbm.at[wid], idx_v)
        base = wid * w
        for p in range(2):
            pltpu.async_copy(rows_hbm.at[pl.ds(base + p * CH, CH)],
                             bufs[p], lsems[p])

        def step(j2, carry):
            for p in range(2):
                j = 2 * j2 + p
                pltpu.make_async_copy(rows_hbm.at[pl.ds(base, CH)],
                                      bufs[p], lsems[p]).wait()
                pltpu.async_copy(bufs[p], acc.at[idx_v.at[j]],
                                 ssems[p], add=True)
                pltpu.make_async_copy(bufs[p], acc.at[pl.ds(0, CH)],
                                      ssems[p]).wait()

                @pl.when(j + 2 < nch)
                def _():
                    pltpu.async_copy(rows_hbm.at[pl.ds(base + (j + 2) * CH, CH)],
                                     bufs[p], lsems[p])
            return carry

        lax.fori_loop(0, nch // 2, step, 0)
        plsc.subcore_barrier()
        for r in range(SLAB // CH):
            pltpu.sync_copy(acc.at[pl.ds(s * SLAB + r * CH, CH)], b0)
            pltpu.sync_copy(b0, out_hbm.at[c, pl.ds(s * SLAB + r * CH, CH)])

    return body


def _make_scatter_kernel(w, with_dep):
    return pl.kernel(
        _make_scatter_body(w, with_dep),
        out_type=jax.ShapeDtypeStruct((NC, N_PAD, D), jnp.float32),
        mesh=_sc_mesh(),
        scratch_types=[
            pltpu.VMEM((w // CH, CH), jnp.int32),
            pltpu.VMEM((CH, D), jnp.float32),
            pltpu.VMEM((CH, D), jnp.float32),
            pltpu.SemaphoreType.DMA,
            pltpu.SemaphoreType.DMA,
            pltpu.SemaphoreType.DMA,
            pltpu.SemaphoreType.DMA,
            pltpu.VMEM_SHARED((N_PAD, D), jnp.float32),
        ],
    )


def _k2_gather(xl_hbm, xr_hbm, src_hbm, dst_hbm, xlg_hbm, xrg_hbm,
               si_v, di_v, a0, a1, b0, b1,
               ga0, ga1, gb0, gb1, wa0, wa1, wb0, wb1):
    c = lax.axis_index("c")
    s = lax.axis_index("s")
    wid = s * NC + c
    abufs = (a0, a1)
    bbufs = (b0, b1)
    gsa = (ga0, ga1)
    gsb = (gb0, gb1)
    wsa = (wa0, wa1)
    wsb = (wb0, wb1)
    base = wid * W2
    pltpu.sync_copy(src_hbm.at[wid], si_v)
    pltpu.sync_copy(dst_hbm.at[wid], di_v)
    for p in range(2):
        pltpu.async_copy(xl_hbm.at[si_v.at[p]], abufs[p], gsa[p])
        pltpu.async_copy(xr_hbm.at[di_v.at[p]], bbufs[p], gsb[p])

    def step(j2, carry):
        for p in range(2):
            j = 2 * j2 + p
            off = base + j * CH
            pltpu.make_async_copy(xl_hbm.at[si_v.at[0]], abufs[p], gsa[p]).wait()
            pltpu.make_async_copy(xr_hbm.at[di_v.at[0]], bbufs[p], gsb[p]).wait()
            pltpu.async_copy(abufs[p], xlg_hbm.at[pl.ds(off, CH)], wsa[p])
            pltpu.async_copy(bbufs[p], xrg_hbm.at[pl.ds(off, CH)], wsb[p])
            pltpu.make_async_copy(abufs[p], xlg_hbm.at[pl.ds(base, CH)], wsa[p]).wait()
            pltpu.make_async_copy(bbufs[p], xrg_hbm.at[pl.ds(base, CH)], wsb[p]).wait()

            @pl.when(j + 2 < NCH2)
            def _():
                pltpu.async_copy(xl_hbm.at[si_v.at[j + 2]], abufs[p], gsa[p])
                pltpu.async_copy(xr_hbm.at[di_v.at[j + 2]], bbufs[p], gsb[p])
        return carry

    lax.fori_loop(0, NCH2 // 2, step, 0)


@functools.lru_cache(maxsize=None)
def _sc_kernels():
    k1 = _make_scatter_kernel(W1, with_dep=False)
    k2 = pl.kernel(
        _k2_gather,
        out_type=(
            jax.ShapeDtypeStruct((EF_PAD, D), jnp.float32),
            jax.ShapeDtypeStruct((EF_PAD, D), jnp.float32),
        ),
        mesh=_sc_mesh(),
        scratch_types=(
            [pltpu.VMEM((NCH2, CH), jnp.int32)] * 2
            + [pltpu.VMEM((CH, D), jnp.float32)] * 4
            + [pltpu.SemaphoreType.DMA] * 8
        ),
    )
    k4a = _make_scatter_kernel(W2, with_dep=False)
    k4b = _make_scatter_kernel(W2, with_dep=True)
    return k1, k2, k4a, k4b


def _k3_body(xlg_ref, xrg_ref, ea8_ref, we8_ref, att_ref, sh_ref, ts_ref, cm_ref,
             contrib_ref, t128_ref):
    i = pl.program_id(0)
    xlg = xlg_ref[...]
    z = xlg + xrg_ref[...] + jnp.dot(ea8_ref[...], we8_ref[...],
                                     preferred_element_type=jnp.float32)
    m = jnp.maximum(z, 0.2 * z)
    p = m * att_ref[...]
    alpha16 = jnp.dot(p, sh_ref[...], preferred_element_type=jnp.float32)
    rows = i * 1024 + lax.broadcasted_iota(jnp.int32, (1024, 1), 0)
    valid = (rows < EF).astype(jnp.float32)
    t16 = jnp.exp(alpha16) * cm_ref[...] * valid
    t128 = jnp.dot(t16, ts_ref[...], preferred_element_type=jnp.float32)
    contrib_ref[...] = xlg * t128
    t128_ref[...] = t128


def _edge_math(xlg, xrg, eaf8, We8, att128, sh, ts, cmask):
    bk = 1024
    return pl.pallas_call(
        _k3_body,
        grid=(EF_PAD // bk,),
        in_specs=[
            pl.BlockSpec((bk, D), lambda i: (i, 0)),
            pl.BlockSpec((bk, D), lambda i: (i, 0)),
            pl.BlockSpec((bk, 8), lambda i: (i, 0)),
            pl.BlockSpec((8, D), lambda i: (0, 0)),
            pl.BlockSpec((1, D), lambda i: (0, 0)),
            pl.BlockSpec((D, 16), lambda i: (0, 0)),
            pl.BlockSpec((16, D), lambda i: (0, 0)),
            pl.BlockSpec((1, 16), lambda i: (0, 0)),
        ],
        out_specs=[
            pl.BlockSpec((bk, D), lambda i: (i, 0)),
            pl.BlockSpec((bk, D), lambda i: (i, 0)),
        ],
        out_shape=[
            jax.ShapeDtypeStruct((EF_PAD, D), jnp.float32),
            jax.ShapeDtypeStruct((EF_PAD, D), jnp.float32),
        ],
    )(xlg, xrg, eaf8, We8, att128, sh, ts, cmask)


def _k5_body(pa0_ref, pa1_ref, pb0_ref, pb1_ref, batch_ref, bias_ref,
             gs_ref, gc_ref):
    i = pl.program_id(0)
    num = pa0_ref[...] + pa1_ref[...]
    den = pb0_ref[...] + pb1_ref[...]
    node = jnp.maximum(num / (den + 1e-16) + bias_ref[...], 0.0)
    b = batch_ref[0]
    oh = (lax.broadcasted_iota(jnp.int32, (G, 400), 0) == b).astype(jnp.float32)
    gsb = jnp.dot(oh, node, preferred_element_type=jnp.float32)
    gcb = jnp.broadcast_to(jnp.sum(oh, axis=1, keepdims=True), (G, D))

    @pl.when(i == 0)
    def _():
        gs_ref[...] = jnp.zeros_like(gs_ref)
        gc_ref[...] = jnp.zeros_like(gc_ref)

    gs_ref[...] += gsb
    gc_ref[...] += gcb


def _pool(pa, pb, batch, bias):
    bk = 400
    return pl.pallas_call(
        _k5_body,
        grid=(N // bk,),
        in_specs=[
            pl.BlockSpec((bk, D), lambda i: (i, 0)),
            pl.BlockSpec((bk, D), lambda i: (i, 0)),
            pl.BlockSpec((bk, D), lambda i: (i, 0)),
            pl.BlockSpec((bk, D), lambda i: (i, 0)),
            pl.BlockSpec((1, 1, bk), lambda i: (i, 0, 0)),
            pl.BlockSpec((1, D), lambda i: (0, 0)),
        ],
        out_specs=[
            pl.BlockSpec((G, D), lambda i: (0, 0)),
            pl.BlockSpec((G, D), lambda i: (0, 0)),
        ],
        out_shape=[
            jax.ShapeDtypeStruct((G, D), jnp.float32),
            jax.ShapeDtypeStruct((G, D), jnp.float32),
        ],
    )(pa[0], pa[1], pb[0], pb[1], batch.reshape(N // bk, 1, bk),
      bias.reshape(1, D))


def kernel(x, edge_index, edge_attr, batch, Wl, bl, Wr, br, We, att, bias):
    f32 = jnp.float32
    src = edge_index[0]
    dst = edge_index[1]

    xl, xr = _dense_proj(x, Wl, bl, Wr, br)
    k1, k2, k4a, k4b = _sc_kernels()
    zn128 = jnp.zeros((CH, D), f32)

    ea128 = jnp.concatenate(
        [edge_attr, jnp.ones((E, 1), f32), jnp.zeros((E, D - ED - 1), f32)], axis=1)
    ea128 = jnp.pad(ea128, ((0, E_PAD - E), (0, 0)))
    spread1 = (jnp.arange(E_PAD - E, dtype=jnp.int32) * 37) % N
    dst_p = jnp.concatenate([dst, spread1]).reshape(NW, NCH1, CH)
    part = k1(ea128, dst_p, zn128)
    s = part[0, :N] + part[1, :N]
    loop_attr = s[:, :ED] / jnp.clip(s[:, ED:ED + 1], 1.0, None)

    loop = jnp.arange(N, dtype=src.dtype)
    spread2 = (jnp.arange(EF_PAD - EF, dtype=jnp.int32) * 37) % N
    srcf = jnp.concatenate([src, loop, spread2]).reshape(NW, NCH2, CH)
    dstf = jnp.concatenate([dst, loop, spread2]).reshape(NW, NCH2, CH)
    xlg, xrg = k2(xl, xr, srcf, dstf)

    eaf8 = jnp.pad(jnp.concatenate([edge_attr, loop_attr], axis=0),
                   ((0, EF_PAD - EF), (0, 4)))
    We8 = jnp.pad(We, ((0, 4), (0, 0)))
    att128 = att.reshape(1, D)
    contrib, t128 = _edge_math(xlg, xrg, eaf8, We8, att128,
                               jnp.asarray(_SH), jnp.asarray(_TS),
                               jnp.asarray(_CMASK))

    pa = k4a(contrib, dstf, zn128)
    pb = k4b(t128, dstf, zn128, pa)

    gs, gc = _pool(pa, pb, batch, bias)
    return gs / jnp.clip(gc, 1.0, None)


# speedup vs baseline: 1.5519x; 1.0011x over previous
import functools

import jax
import jax.numpy as jnp
import numpy as np
from jax import lax
from jax.experimental import pallas as pl
from jax.experimental.pallas import tpu as pltpu
from jax.experimental.pallas import tpu_sc as plsc

N = 10000
E = 320000
IN = 128
H = 4
C = 32
ED = 4
G = 16
D = H * C
EF = E + N

NC = 2
NS = 16
NW = NC * NS
CH = 128
N_PAD = 10240
SLAB = N_PAD // NS

W1 = 10496
E_PAD = W1 * NW
W2 = 10752
EF_PAD = W2 * NW
NCH1 = W1 // CH
NCH2 = W2 // CH


def _sc_mesh():
    return plsc.VectorSubcoreMesh(core_axis_name="c", subcore_axis_name="s",
                                  num_cores=NC, num_subcores=NS)


_SH = np.zeros((D, 16), np.float32)
_TS = np.zeros((16, D), np.float32)
for _c in range(D):
    _SH[_c, _c // C] = 1.0
    _TS[_c // C, _c] = 1.0
_CMASK = np.zeros((1, 16), np.float32)
_CMASK[0, :H] = 1.0


def _k0_body(x_ref, wl_ref, bl_ref, wr_ref, br_ref, xl_ref, xr_ref):
    xb = x_ref[...]
    xl_ref[...] = jnp.dot(xb, wl_ref[...], preferred_element_type=jnp.float32) + bl_ref[...]
    xr_ref[...] = jnp.dot(xb, wr_ref[...], preferred_element_type=jnp.float32) + br_ref[...]


def _dense_proj(x, Wl, bl, Wr, br):
    bk = 1000
    return pl.pallas_call(
        _k0_body,
        grid=(N // bk,),
        in_specs=[
            pl.BlockSpec((bk, IN), lambda i: (i, 0)),
            pl.BlockSpec((IN, D), lambda i: (0, 0)),
            pl.BlockSpec((1, D), lambda i: (0, 0)),
            pl.BlockSpec((IN, D), lambda i: (0, 0)),
            pl.BlockSpec((1, D), lambda i: (0, 0)),
        ],
        out_specs=[
            pl.BlockSpec((bk, D), lambda i: (i, 0)),
            pl.BlockSpec((bk, D), lambda i: (i, 0)),
        ],
        out_shape=[
            jax.ShapeDtypeStruct((N, D), jnp.float32),
            jax.ShapeDtypeStruct((N, D), jnp.float32),
        ],
    )(x, Wl, bl.reshape(1, D), Wr, br.reshape(1, D))


def _make_scatter_body(w, with_dep):
    nch = w // CH

    def body(*refs):
        if with_dep:
            (rows_hbm, dst_hbm, zn_hbm, _dep, out_hbm,
             idx_v, b0, b1, ls0, ls1, ss0, ss1, acc) = refs
        else:
            (rows_hbm, dst_hbm, zn_hbm, out_hbm,
             idx_v, b0, b1, ls0, ls1, ss0, ss1, acc) = refs
        c = lax.axis_index("c")
        s = lax.axis_index("s")
        wid = s * NC + c
        bufs = (b0, b1)
        lsems = (ls0, ls1)
        ssems = (ss0, ss1)
        pltpu.sync_copy(zn_hbm, b0)
        for r in range(SLAB // CH):
            pltpu.sync_copy(b0, acc.at[pl.ds(s * SLAB + r * CH, CH)])
        plsc.subcore_barrier()
        pltpu.sync_copy(dst_hbm.at[wid], idx_v)
        base = wid * w
        for p in range(2):
            pltpu.async_copy(rows_hbm.at[pl.ds(base + p * CH, CH)],
                             bufs[p], lsems[p])

        def step(j2, carry):
            for p in range(2):
                j = 2 * j2 + p
                pltpu.make_async_copy(rows_hbm.at[pl.ds(base, CH)],
                                      bufs[p], lsems[p]).wait()
                pltpu.async_copy(bufs[p], acc.at[idx_v.at[j]],
                                 ssems[p], add=True)
                pltpu.make_async_copy(bufs[p], acc.at[pl.ds(0, CH)],
                                      ssems[p]).wait()

                @pl.when(j + 2 < nch)
                def _():
                    pltpu.async_copy(rows_hbm.at[pl.ds(base + (j + 2) * CH, CH)],
                                     bufs[p], lsems[p])
            return carry

        lax.fori_loop(0, nch // 2, step, 0)
        plsc.subcore_barrier()
        for r in range(SLAB // CH):
            pltpu.sync_copy(acc.at[pl.ds(s * SLAB + r * CH, CH)], b0)
            pltpu.sync_copy(b0, out_hbm.at[c, pl.ds(s * SLAB + r * CH, CH)])

    return body


def _make_scatter_kernel(w, with_dep):
    return pl.kernel(
        _make_scatter_body(w, with_dep),
        out_type=jax.ShapeDtypeStruct((NC, N_PAD, D), jnp.float32),
        mesh=_sc_mesh(),
        scratch_types=[
            pltpu.VMEM((w // CH, CH), jnp.int32),
            pltpu.VMEM((CH, D), jnp.float32),
            pltpu.VMEM((CH, D), jnp.float32),
            pltpu.SemaphoreType.DMA,
            pltpu.SemaphoreType.DMA,
            pltpu.SemaphoreType.DMA,
            pltpu.SemaphoreType.DMA,
            pltpu.VMEM_SHARED((N_PAD, D), jnp.float32),
        ],
    )


def _k2_gather(xl_hbm, xr_hbm, src_hbm, dst_hbm, xlg_hbm, xrg_hbm,
               si_v, di_v, a0, a1, a2, b0, b1, b2,
               ga0, ga1, ga2, gb0, gb1, gb2, wa0, wa1, wa2, wb0, wb1, wb2):
    c = lax.axis_index("c")
    s = lax.axis_index("s")
    wid = s * NC + c
    abufs = (a0, a1, a2)
    bbufs = (b0, b1, b2)
    gsa = (ga0, ga1, ga2)
    gsb = (gb0, gb1, gb2)
    wsa = (wa0, wa1, wa2)
    wsb = (wb0, wb1, wb2)
    base = wid * W2
    pltpu.sync_copy(src_hbm.at[wid], si_v)
    pltpu.sync_copy(dst_hbm.at[wid], di_v)
    for p in range(2):
        pltpu.async_copy(xl_hbm.at[si_v.at[p]], abufs[p], gsa[p])
        pltpu.async_copy(xr_hbm.at[di_v.at[p]], bbufs[p], gsb[p])

    def step(j3, carry):
        for p in range(3):
            j = 3 * j3 + p
            pn = (p + 2) % 3
            off = base + j * CH
            pltpu.make_async_copy(xl_hbm.at[si_v.at[0]], abufs[p], gsa[p]).wait()
            pltpu.make_async_copy(xr_hbm.at[di_v.at[0]], bbufs[p], gsb[p]).wait()
            pltpu.async_copy(abufs[p], xlg_hbm.at[pl.ds(off, CH)], wsa[p])
            pltpu.async_copy(bbufs[p], xrg_hbm.at[pl.ds(off, CH)], wsb[p])

            @pl.when(j + 2 < NCH2)
            def _():
                @pl.when(j >= 1)
                def _():
                    pltpu.make_async_copy(abufs[pn], xlg_hbm.at[pl.ds(base, CH)],
                                          wsa[pn]).wait()
                    pltpu.make_async_copy(bbufs[pn], xrg_hbm.at[pl.ds(base, CH)],
                                          wsb[pn]).wait()
                pltpu.async_copy(xl_hbm.at[si_v.at[j + 2]], abufs[pn], gsa[pn])
                pltpu.async_copy(xr_hbm.at[di_v.at[j + 2]], bbufs[pn], gsb[pn])
        return carry

    lax.fori_loop(0, NCH2 // 3, step, 0)
    for p in range(3):
        pltpu.make_async_copy(abufs[p], xlg_hbm.at[pl.ds(base, CH)], wsa[p]).wait()
        pltpu.make_async_copy(bbufs[p], xrg_hbm.at[pl.ds(base, CH)], wsb[p]).wait()


@functools.lru_cache(maxsize=None)
def _sc_kernels():
    k1 = _make_scatter_kernel(W1, with_dep=False)
    k2 = pl.kernel(
        _k2_gather,
        out_type=(
            jax.ShapeDtypeStruct((EF_PAD, D), jnp.float32),
            jax.ShapeDtypeStruct((EF_PAD, D), jnp.float32),
        ),
        mesh=_sc_mesh(),
        scratch_types=(
            [pltpu.VMEM((NCH2, CH), jnp.int32)] * 2
            + [pltpu.VMEM((CH, D), jnp.float32)] * 6
            + [pltpu.SemaphoreType.DMA] * 12
        ),
    )
    k4a = _make_scatter_kernel(W2, with_dep=False)
    k4b = _make_scatter_kernel(W2, with_dep=True)
    return k1, k2, k4a, k4b


def _k3_body(xlg_ref, xrg_ref, ea8_ref, we8_ref, att_ref, sh_ref, ts_ref, cm_ref,
             contrib_ref, t128_ref):
    i = pl.program_id(0)
    xlg = xlg_ref[...]
    z = xlg + xrg_ref[...] + jnp.dot(ea8_ref[...], we8_ref[...],
                                     preferred_element_type=jnp.float32)
    m = jnp.maximum(z, 0.2 * z)
    p = m * att_ref[...]
    alpha16 = jnp.dot(p, sh_ref[...], preferred_element_type=jnp.float32)
    rows = i * 1024 + lax.broadcasted_iota(jnp.int32, (1024, 1), 0)
    valid = (rows < EF).astype(jnp.float32)
    t16 = jnp.exp(alpha16) * cm_ref[...] * valid
    t128 = jnp.dot(t16, ts_ref[...], preferred_element_type=jnp.float32)
    contrib_ref[...] = xlg * t128
    t128_ref[...] = t128


def _edge_math(xlg, xrg, eaf8, We8, att128, sh, ts, cmask):
    bk = 1024
    return pl.pallas_call(
        _k3_body,
        grid=(EF_PAD // bk,),
        in_specs=[
            pl.BlockSpec((bk, D), lambda i: (i, 0)),
            pl.BlockSpec((bk, D), lambda i: (i, 0)),
            pl.BlockSpec((bk, 8), lambda i: (i, 0)),
            pl.BlockSpec((8, D), lambda i: (0, 0)),
            pl.BlockSpec((1, D), lambda i: (0, 0)),
            pl.BlockSpec((D, 16), lambda i: (0, 0)),
            pl.BlockSpec((16, D), lambda i: (0, 0)),
            pl.BlockSpec((1, 16), lambda i: (0, 0)),
        ],
        out_specs=[
            pl.BlockSpec((bk, D), lambda i: (i, 0)),
            pl.BlockSpec((bk, D), lambda i: (i, 0)),
        ],
        out_shape=[
            jax.ShapeDtypeStruct((EF_PAD, D), jnp.float32),
            jax.ShapeDtypeStruct((EF_PAD, D), jnp.float32),
        ],
    )(xlg, xrg, eaf8, We8, att128, sh, ts, cmask)


def _k5_body(pa0_ref, pa1_ref, pb0_ref, pb1_ref, batch_ref, bias_ref,
             gs_ref, gc_ref):
    i = pl.program_id(0)
    num = pa0_ref[...] + pa1_ref[...]
    den = pb0_ref[...] + pb1_ref[...]
    node = jnp.maximum(num / (den + 1e-16) + bias_ref[...], 0.0)
    b = batch_ref[0]
    oh = (lax.broadcasted_iota(jnp.int32, (G, 400), 0) == b).astype(jnp.float32)
    gsb = jnp.dot(oh, node, preferred_element_type=jnp.float32)
    gcb = jnp.broadcast_to(jnp.sum(oh, axis=1, keepdims=True), (G, D))

    @pl.when(i == 0)
    def _():
        gs_ref[...] = jnp.zeros_like(gs_ref)
        gc_ref[...] = jnp.zeros_like(gc_ref)

    gs_ref[...] += gsb
    gc_ref[...] += gcb


def _pool(pa, pb, batch, bias):
    bk = 400
    return pl.pallas_call(
        _k5_body,
        grid=(N // bk,),
        in_specs=[
            pl.BlockSpec((bk, D), lambda i: (i, 0)),
            pl.BlockSpec((bk, D), lambda i: (i, 0)),
            pl.BlockSpec((bk, D), lambda i: (i, 0)),
            pl.BlockSpec((bk, D), lambda i: (i, 0)),
            pl.BlockSpec((1, 1, bk), lambda i: (i, 0, 0)),
            pl.BlockSpec((1, D), lambda i: (0, 0)),
        ],
        out_specs=[
            pl.BlockSpec((G, D), lambda i: (0, 0)),
            pl.BlockSpec((G, D), lambda i: (0, 0)),
        ],
        out_shape=[
            jax.ShapeDtypeStruct((G, D), jnp.float32),
            jax.ShapeDtypeStruct((G, D), jnp.float32),
        ],
    )(pa[0], pa[1], pb[0], pb[1], batch.reshape(N // bk, 1, bk),
      bias.reshape(1, D))


def kernel(x, edge_index, edge_attr, batch, Wl, bl, Wr, br, We, att, bias):
    f32 = jnp.float32
    src = edge_index[0]
    dst = edge_index[1]

    xl, xr = _dense_proj(x, Wl, bl, Wr, br)
    k1, k2, k4a, k4b = _sc_kernels()
    zn128 = jnp.zeros((CH, D), f32)

    ea128 = jnp.concatenate(
        [edge_attr, jnp.ones((E, 1), f32), jnp.zeros((E, D - ED - 1), f32)], axis=1)
    ea128 = jnp.pad(ea128, ((0, E_PAD - E), (0, 0)))
    spread1 = (jnp.arange(E_PAD - E, dtype=jnp.int32) * 37) % N
    dst_p = jnp.concatenate([dst, spread1]).reshape(NW, NCH1, CH)
    part = k1(ea128, dst_p, zn128)
    s = part[0, :N] + part[1, :N]
    loop_attr = s[:, :ED] / jnp.clip(s[:, ED:ED + 1], 1.0, None)

    loop = jnp.arange(N, dtype=src.dtype)
    spread2 = (jnp.arange(EF_PAD - EF, dtype=jnp.int32) * 37) % N
    srcf = jnp.concatenate([src, loop, spread2]).reshape(NW, NCH2, CH)
    dstf = jnp.concatenate([dst, loop, spread2]).reshape(NW, NCH2, CH)
    xlg, xrg = k2(xl, xr, srcf, dstf)

    eaf8 = jnp.pad(jnp.concatenate([edge_attr, loop_attr], axis=0),
                   ((0, EF_PAD - EF), (0, 4)))
    We8 = jnp.pad(We, ((0, 4), (0, 0)))
    att128 = att.reshape(1, D)
    contrib, t128 = _edge_math(xlg, xrg, eaf8, We8, att128,
                               jnp.asarray(_SH), jnp.asarray(_TS),
                               jnp.asarray(_CMASK))

    pa = k4a(contrib, dstf, zn128)
    pb = k4b(t128, dstf, zn128, pa)

    gs, gc = _pool(pa, pb, batch, bias)
    return gs / jnp.clip(gc, 1.0, None)
